# baseline (speedup 1.0000x reference)
import functools

import jax
import jax.numpy as jnp
from jax import lax
from jax.experimental import pallas as pl
from jax.experimental.pallas import tpu as pltpu
from jax.experimental.pallas import tpu_sc as plsc

N = 10000
E = 320000
D = 128
G = 32
NC = 2
NS = 16
N_PAD = 10240
ROWS_PER_TILE = N_PAD // NS
CHUNK = 128
NCHUNK = 80
NGRP = 2
GCH = NCHUNK // NGRP
NSUB = 4
REP = 2
E_PAD = NC * NS * NCHUNK * CHUNK
PAD_E = E_PAD - E

_F32 = jnp.float32


def _make_agg(with_cnt: bool):
    mesh = plsc.VectorSubcoreMesh(core_axis_name="c", subcore_axis_name="s")
    if with_cnt:
        out_type = [jax.ShapeDtypeStruct((NC, N_PAD, D), _F32),
                    jax.ShapeDtypeStruct((NC, N_PAD), _F32)]
    else:
        out_type = jax.ShapeDtypeStruct((NC, N_PAD, D), _F32)
    scratch = [
        pltpu.VMEM((GCH, CHUNK), jnp.int32),
        pltpu.VMEM((GCH, CHUNK), jnp.int32),
        pltpu.VMEM((CHUNK, D), _F32),
        pltpu.VMEM((CHUNK, D), _F32),
        pltpu.VMEM((CHUNK,), _F32),
        pltpu.VMEM_SHARED((N_PAD, D), _F32),
        pltpu.VMEM_SHARED((N_PAD,), _F32),
        pltpu.SemaphoreType.DMA,
        pltpu.SemaphoreType.DMA,
    ]

    def body(feats_hbm, src_hbm, dst_hbm, zrows_hbm, zcnt_hbm, *rest):
        if with_cnt:
            out_agg, out_cnt = rest[0], rest[1]
            rest = rest[2:]
        else:
            out_agg = rest[0]
            rest = rest[1:]
        (idx_s, idx_d, rows0, rows1, ones_v, agg_sh, cnt_sh,
         sem0, sem1) = rest
        cid = lax.axis_index("c")
        sid = lax.axis_index("s")
        row0 = sid * ROWS_PER_TILE
        pltpu.sync_copy(zrows_hbm, agg_sh.at[pl.ds(row0, ROWS_PER_TILE)])
        if with_cnt:
            @pl.when(sid == 0)
            def _():
                pltpu.sync_copy(zcnt_hbm, cnt_sh)
            for k in range(CHUNK // 16):
                ones_v[pl.ds(k * 16, 16)] = jnp.ones((16,), _F32)
        plsc.subcore_barrier()

        SUB = CHUNK // NSUB

        def gather_start(i, buf, sem):
            for s in range(NSUB):
                pltpu.async_copy(
                    feats_hbm.at[idx_s.at[i, pl.ds(s * SUB, SUB)]],
                    buf.at[pl.ds(s * SUB, SUB)], sem)

        def gather_wait(i, buf, sem):
            for s in range(NSUB):
                pltpu.make_async_copy(
                    feats_hbm.at[idx_s.at[i, pl.ds(s * SUB, SUB)]],
                    buf.at[pl.ds(s * SUB, SUB)], sem).wait()

        def scatter(i, buf):
            pltpu.sync_copy(buf, agg_sh.at[idx_d.at[i]], add=True)
            if with_cnt:
                pltpu.sync_copy(ones_v, cnt_sh.at[idx_d.at[i]], add=True)

        for g in range(NGRP):
            pltpu.sync_copy(src_hbm.at[cid, sid, pl.ds(g * GCH, GCH)], idx_s)
            pltpu.sync_copy(dst_hbm.at[cid, sid, pl.ds(g * GCH, GCH)], idx_d)
            gather_start(0, rows0, sem0)

            def step(j, carry):
                i0 = j * 2
                gather_wait(i0, rows0, sem0)
                gather_start(i0 + 1, rows1, sem1)
                scatter(i0, rows0)
                gather_wait(i0 + 1, rows1, sem1)
                gather_start(i0 + 2, rows0, sem0)
                scatter(i0 + 1, rows1)
                return carry

            lax.fori_loop(0, GCH // 2 - 1, step, 0)
            i0 = GCH - 2
            gather_wait(i0, rows0, sem0)
            gather_start(i0 + 1, rows1, sem1)
            scatter(i0, rows0)
            gather_wait(i0 + 1, rows1, sem1)
            scatter(i0 + 1, rows1)
        plsc.subcore_barrier()
        pltpu.sync_copy(agg_sh.at[pl.ds(row0, ROWS_PER_TILE)],
                        out_agg.at[cid, pl.ds(row0, ROWS_PER_TILE)])
        if with_cnt:
            @pl.when(sid == 0)
            def _():
                pltpu.sync_copy(cnt_sh, out_cnt.at[cid])

    return pl.kernel(body, mesh=mesh, out_type=out_type, scratch_types=scratch)


_agg_with_cnt = _make_agg(True)
_agg = _make_agg(False)


R_UPD = 2000


def _update_body(p_ref, c_ref, h_ref, wl_ref, wr_ref, b_ref, o_ref):
    cnt = c_ref[0] + c_ref[1]
    inv = 1.0 / jnp.maximum(cnt, 1.0)
    mean = (p_ref[0] + p_ref[1]) * inv
    dn = (((1,), (1,)), ((), ()))
    acc = lax.dot_general(mean, wl_ref[...], dn,
                          preferred_element_type=_F32,
                          precision=lax.Precision.HIGHEST)
    acc = acc + lax.dot_general(h_ref[0], wr_ref[...], dn,
                                preferred_element_type=_F32,
                                precision=lax.Precision.HIGHEST)
    out = jnp.maximum(acc + b_ref[...], 0.0)
    for k in range(REP):
        o_ref[k] = out


def _update(p, cnt2, h, Wl, Wr, b):
    grid = N // R_UPD
    return pl.pallas_call(
        _update_body,
        grid=(grid,),
        in_specs=[
            pl.BlockSpec((NC, R_UPD, D), lambda i: (0, i, 0)),
            pl.BlockSpec((NC, R_UPD, 1), lambda i: (0, i, 0)),
            pl.BlockSpec((1, R_UPD, D), lambda i: (0, i, 0)),
            pl.BlockSpec((D, D), lambda i: (0, 0)),
            pl.BlockSpec((D, D), lambda i: (0, 0)),
            pl.BlockSpec((1, D), lambda i: (0, 0)),
        ],
        out_specs=pl.BlockSpec((REP, R_UPD, D), lambda i: (0, i, 0)),
        out_shape=jax.ShapeDtypeStruct((REP, N, D), _F32),
    )(p, cnt2, h, Wl, Wr, b)


NEG = -3.0e38


def _fused3_body(p_ref, c_ref, h_ref, wl_ref, wr_ref, b_ref,
                 x_ref, bcol_ref, brow_ref, srow_ref,
                 f1w_ref, f1b_ref, f2w_ref, f2b_ref, smw_ref, smb_ref,
                 nw_ref, nb_ref, cw_ref, cb_ref, o_ref,
                 pooled_ref, news_ref):
    i = pl.program_id(0)

    @pl.when(i == 0)
    def _():
        pooled_ref[...] = jnp.full((G, D), NEG, _F32)
        news_ref[...] = jnp.zeros((G, D), _F32)

    cnt = c_ref[0] + c_ref[1]
    inv = 1.0 / jnp.maximum(cnt, 1.0)
    mean = (p_ref[0] + p_ref[1]) * inv
    dn = (((1,), (1,)), ((), ()))
    acc = lax.dot_general(mean, wl_ref[...], dn,
                          preferred_element_type=_F32,
                          precision=lax.Precision.HIGHEST)
    acc = acc + lax.dot_general(h_ref[0], wr_ref[...], dn,
                                preferred_element_type=_F32,
                                precision=lax.Precision.HIGHEST)
    hb = jnp.maximum(acc + b_ref[...], 0.0)

    bb = bcol_ref[...]
    blockmax = jnp.concatenate(
        [jnp.max(jnp.where(bb == float(g), hb, NEG), axis=0, keepdims=True)
         for g in range(G)], axis=0)
    pooled_ref[...] = jnp.maximum(pooled_ref[...], blockmax)

    br = brow_ref[0]
    sr = srow_ref[0]
    isroot = jnp.where(br != sr, 1.0, 0.0)
    gid = lax.broadcasted_iota(jnp.int32, (G, 1), 0).astype(_F32)
    onehot = jnp.where(br == gid, 1.0, 0.0) * isroot
    news_ref[...] += lax.dot_general(
        onehot, x_ref[...], (((1,), (0,)), ((), ())),
        preferred_element_type=_F32, precision=lax.Precision.HIGHEST)

    @pl.when(i == (N // R_UPD) - 1)
    def _():
        dn = (((1,), (1,)), ((), ()))

        def dense(v, w, b):
            return lax.dot_general(v, w, dn, preferred_element_type=_F32,
                                   precision=lax.Precision.HIGHEST) + b

        h1 = jnp.maximum(dense(pooled_ref[...], f1w_ref[...], f1b_ref[...]), 0.0)
        h2 = jnp.maximum(dense(h1, f2w_ref[...], f2b_ref[...]), 0.0)
        h3 = jnp.maximum(dense(h2, smw_ref[...], smb_ref[...]), 0.0)
        nl = jnp.maximum(dense(news_ref[...], nw_ref[...], nb_ref[...]), 0.0)
        cw = cw_ref[...]
        logit = (jnp.sum(h3 * cw[:, :64], axis=1, keepdims=True)
                 + jnp.sum(nl * cw[:, 64:], axis=1, keepdims=True)
                 + cb_ref[...])
        o_ref[...] = 1.0 / (1.0 + jnp.exp(-logit))


def _fused3(p, cnt2, h, Wl, Wr, b, x, bcol, brow, srow,
            f1w, f1b, f2w, f2b, smw, smb, nw, nb, cw, cb):
    grid = N // R_UPD
    full = lambda shape: pl.BlockSpec(shape, lambda i: tuple(0 for _ in shape))
    return pl.pallas_call(
        _fused3_body,
        grid=(grid,),
        in_specs=[
            pl.BlockSpec((NC, R_UPD, D), lambda i: (0, i, 0)),
            pl.BlockSpec((NC, R_UPD, 1), lambda i: (0, i, 0)),
            pl.BlockSpec((1, R_UPD, D), lambda i: (0, i, 0)),
            full((D, D)), full((D, D)), full((1, D)),
            pl.BlockSpec((R_UPD, D), lambda i: (i, 0)),
            pl.BlockSpec((R_UPD, 1), lambda i: (i, 0)),
            pl.BlockSpec((1, 1, R_UPD), lambda i: (i, 0, 0)),
            pl.BlockSpec((1, 1, R_UPD), lambda i: (i, 0, 0)),
            full((D, D)), full((1, D)),
            full((64, D)), full((1, 64)),
            full((64, 64)), full((1, 64)),
            full((64, D)), full((1, 64)),
            full((1, D)), full((1, 1)),
        ],
        out_specs=pl.BlockSpec((G, 1), lambda i: (0, 0)),
        out_shape=jax.ShapeDtypeStruct((G, 1), _F32),
        scratch_shapes=[pltpu.VMEM((G, D), _F32), pltpu.VMEM((G, D), _F32)],
    )(p, cnt2, h, Wl, Wr, b, x, bcol, brow, srow,
      f1w, f1b, f2w, f2b, smw, smb, nw, nb, cw, cb)


def kernel(x, edge_index, batch,
           conv1_Wl, conv1_Wr, conv1_b,
           conv2_Wl, conv2_Wr, conv2_b,
           conv3_Wl, conv3_Wr, conv3_b,
           full1_W, full1_b, full2_W, full2_b,
           softmax_W, softmax_b, lin_news_W, lin_news_b,
           lin_cat_W, lin_cat_b):
    pad_iota = jnp.arange(PAD_E, dtype=jnp.int32)
    src = jnp.concatenate([edge_index[0], pad_iota % N])
    src = src.reshape(NC, NS, NCHUNK, CHUNK)
    rep_off = (jnp.arange(NCHUNK, dtype=jnp.int32) % REP) * N
    src = src + rep_off[None, None, :, None]
    dst = jnp.concatenate([edge_index[1], N + pad_iota % (N_PAD - N)])
    dst = dst.reshape(NC, NS, NCHUNK, CHUNK)
    zrows = jnp.zeros((ROWS_PER_TILE, D), _F32)
    zcnt = jnp.zeros((N_PAD,), _F32)

    xrep = jnp.broadcast_to(x[None], (REP, N, D))
    p1, cnt = _agg_with_cnt(xrep.reshape(REP * N, D), src, dst, zrows, zcnt)
    cnt2 = cnt.reshape(NC, N_PAD, 1)
    h1 = _update(p1, cnt2, xrep, conv1_Wl, conv1_Wr, conv1_b.reshape(1, D))
    p2 = _agg(h1.reshape(REP * N, D), src, dst, zrows, zcnt)
    h2 = _update(p2, cnt2, h1, conv2_Wl, conv2_Wr, conv2_b.reshape(1, D))
    p3 = _agg(h2.reshape(REP * N, D), src, dst, zrows, zcnt)

    bf = batch.astype(_F32)
    bcol = bf.reshape(N, 1)
    flat = bf.reshape(1, N)
    sflat = jnp.concatenate([jnp.full((1, 1), -1.0, _F32), flat[:, :-1]], axis=1)
    brow = flat.reshape(N // R_UPD, 1, R_UPD)
    srow = sflat.reshape(N // R_UPD, 1, R_UPD)
    return _fused3(p3, cnt2, h2, conv3_Wl, conv3_Wr, conv3_b.reshape(1, D),
                   x, bcol, brow, srow,
                   full1_W, full1_b.reshape(1, D),
                   full2_W, full2_b.reshape(1, 64),
                   softmax_W, softmax_b.reshape(1, 64),
                   lin_news_W, lin_news_b.reshape(1, 64),
                   lin_cat_W, lin_cat_b.reshape(1, 1))


# speedup vs baseline: 1.0071x; 1.0071x over previous
import functools

import jax
import jax.numpy as jnp
from jax import lax
from jax.experimental import pallas as pl
from jax.experimental.pallas import tpu as pltpu
from jax.experimental.pallas import tpu_sc as plsc

N = 10000
E = 320000
D = 128
G = 32
NC = 2
NS = 16
N_PAD = 10240
ROWS_PER_TILE = N_PAD // NS
CHUNK = 128
NCHUNK = 80
NGRP = 2
GCH = NCHUNK // NGRP
NSUB = 1
E_PAD = NC * NS * NCHUNK * CHUNK
PAD_E = E_PAD - E

_F32 = jnp.float32


def _make_agg(with_cnt: bool):
    mesh = plsc.VectorSubcoreMesh(core_axis_name="c", subcore_axis_name="s")
    if with_cnt:
        out_type = [jax.ShapeDtypeStruct((NC, N_PAD, D), _F32),
                    jax.ShapeDtypeStruct((NC, N_PAD), _F32)]
    else:
        out_type = jax.ShapeDtypeStruct((NC, N_PAD, D), _F32)
    scratch = [
        pltpu.VMEM((GCH, CHUNK), jnp.int32),
        pltpu.VMEM((GCH, CHUNK), jnp.int32),
        pltpu.VMEM((CHUNK, D), _F32),
        pltpu.VMEM((CHUNK, D), _F32),
        pltpu.VMEM((CHUNK,), _F32),
        pltpu.VMEM_SHARED((N_PAD, D), _F32),
        pltpu.VMEM_SHARED((N_PAD,), _F32),
        pltpu.SemaphoreType.DMA,
        pltpu.SemaphoreType.DMA,
    ]

    def body(feats_hbm, src_hbm, dst_hbm, zrows_hbm, zcnt_hbm, *rest):
        if with_cnt:
            out_agg, out_cnt = rest[0], rest[1]
            rest = rest[2:]
        else:
            out_agg = rest[0]
            rest = rest[1:]
        (idx_s, idx_d, rows0, rows1, ones_v, agg_sh, cnt_sh,
         sem0, sem1) = rest
        cid = lax.axis_index("c")
        sid = lax.axis_index("s")
        row0 = sid * ROWS_PER_TILE
        pltpu.sync_copy(zrows_hbm, agg_sh.at[pl.ds(row0, ROWS_PER_TILE)])
        if with_cnt:
            @pl.when(sid == 0)
            def _():
                pltpu.sync_copy(zcnt_hbm, cnt_sh)
            for k in range(CHUNK // 16):
                ones_v[pl.ds(k * 16, 16)] = jnp.ones((16,), _F32)
        plsc.subcore_barrier()

        SUB = CHUNK // NSUB

        def gather_start(i, buf, sem):
            for s in range(NSUB):
                pltpu.async_copy(
                    feats_hbm.at[idx_s.at[i, pl.ds(s * SUB, SUB)]],
                    buf.at[pl.ds(s * SUB, SUB)], sem)

        def gather_wait(i, buf, sem):
            for s in range(NSUB):
                pltpu.make_async_copy(
                    feats_hbm.at[idx_s.at[i, pl.ds(s * SUB, SUB)]],
                    buf.at[pl.ds(s * SUB, SUB)], sem).wait()

        def scatter(i, buf):
            pltpu.sync_copy(buf, agg_sh.at[idx_d.at[i]], add=True)
            if with_cnt:
                pltpu.sync_copy(ones_v, cnt_sh.at[idx_d.at[i]], add=True)

        for g in range(NGRP):
            pltpu.sync_copy(src_hbm.at[cid, sid, pl.ds(g * GCH, GCH)], idx_s)
            pltpu.sync_copy(dst_hbm.at[cid, sid, pl.ds(g * GCH, GCH)], idx_d)
            gather_start(0, rows0, sem0)

            def step(j, carry):
                i0 = j * 2
                gather_wait(i0, rows0, sem0)
                gather_start(i0 + 1, rows1, sem1)
                scatter(i0, rows0)
                gather_wait(i0 + 1, rows1, sem1)
                gather_start(i0 + 2, rows0, sem0)
                scatter(i0 + 1, rows1)
                return carry

            lax.fori_loop(0, GCH // 2 - 1, step, 0)
            i0 = GCH - 2
            gather_wait(i0, rows0, sem0)
            gather_start(i0 + 1, rows1, sem1)
            scatter(i0, rows0)
            gather_wait(i0 + 1, rows1, sem1)
            scatter(i0 + 1, rows1)
        plsc.subcore_barrier()
        pltpu.sync_copy(agg_sh.at[pl.ds(row0, ROWS_PER_TILE)],
                        out_agg.at[cid, pl.ds(row0, ROWS_PER_TILE)])
        if with_cnt:
            @pl.when(sid == 0)
            def _():
                pltpu.sync_copy(cnt_sh, out_cnt.at[cid])

    return pl.kernel(body, mesh=mesh, out_type=out_type, scratch_types=scratch)


_agg_with_cnt = _make_agg(True)
_agg = _make_agg(False)


R_UPD = 2000


def _update_body(p_ref, c_ref, h_ref, wl_ref, wr_ref, b_ref, o_ref):
    cnt = c_ref[0] + c_ref[1]
    inv = 1.0 / jnp.maximum(cnt, 1.0)
    mean = (p_ref[0] + p_ref[1]) * inv
    dn = (((1,), (1,)), ((), ()))
    acc = lax.dot_general(mean, wl_ref[...], dn,
                          preferred_element_type=_F32,
                          precision=lax.Precision.HIGHEST)
    acc = acc + lax.dot_general(h_ref[...], wr_ref[...], dn,
                                preferred_element_type=_F32,
                                precision=lax.Precision.HIGHEST)
    o_ref[...] = jnp.maximum(acc + b_ref[...], 0.0)


def _update(p, cnt2, h, Wl, Wr, b):
    grid = N // R_UPD
    return pl.pallas_call(
        _update_body,
        grid=(grid,),
        in_specs=[
            pl.BlockSpec((NC, R_UPD, D), lambda i: (0, i, 0)),
            pl.BlockSpec((NC, R_UPD, 1), lambda i: (0, i, 0)),
            pl.BlockSpec((R_UPD, D), lambda i: (i, 0)),
            pl.BlockSpec((D, D), lambda i: (0, 0)),
            pl.BlockSpec((D, D), lambda i: (0, 0)),
            pl.BlockSpec((1, D), lambda i: (0, 0)),
        ],
        out_specs=pl.BlockSpec((R_UPD, D), lambda i: (i, 0)),
        out_shape=jax.ShapeDtypeStruct((N, D), _F32),
    )(p, cnt2, h, Wl, Wr, b)


NEG = -3.0e38


def _fused3_body(p_ref, c_ref, h_ref, wl_ref, wr_ref, b_ref,
                 x_ref, bcol_ref, brow_ref, srow_ref,
                 f1w_ref, f1b_ref, f2w_ref, f2b_ref, smw_ref, smb_ref,
                 nw_ref, nb_ref, cw_ref, cb_ref, o_ref,
                 pooled_ref, news_ref):
    i = pl.program_id(0)

    @pl.when(i == 0)
    def _():
        pooled_ref[...] = jnp.full((G, D), NEG, _F32)
        news_ref[...] = jnp.zeros((G, D), _F32)

    cnt = c_ref[0] + c_ref[1]
    inv = 1.0 / jnp.maximum(cnt, 1.0)
    mean = (p_ref[0] + p_ref[1]) * inv
    dn = (((1,), (1,)), ((), ()))
    acc = lax.dot_general(mean, wl_ref[...], dn,
                          preferred_element_type=_F32,
                          precision=lax.Precision.HIGHEST)
    acc = acc + lax.dot_general(h_ref[...], wr_ref[...], dn,
                                preferred_element_type=_F32,
                                precision=lax.Precision.HIGHEST)
    hb = jnp.maximum(acc + b_ref[...], 0.0)

    bb = bcol_ref[...]
    blockmax = jnp.concatenate(
        [jnp.max(jnp.where(bb == float(g), hb, NEG), axis=0, keepdims=True)
         for g in range(G)], axis=0)
    pooled_ref[...] = jnp.maximum(pooled_ref[...], blockmax)

    br = brow_ref[0]
    sr = srow_ref[0]
    isroot = jnp.where(br != sr, 1.0, 0.0)
    gid = lax.broadcasted_iota(jnp.int32, (G, 1), 0).astype(_F32)
    onehot = jnp.where(br == gid, 1.0, 0.0) * isroot
    news_ref[...] += lax.dot_general(
        onehot, x_ref[...], (((1,), (0,)), ((), ())),
        preferred_element_type=_F32, precision=lax.Precision.HIGHEST)

    @pl.when(i == (N // R_UPD) - 1)
    def _():
        dn = (((1,), (1,)), ((), ()))

        def dense(v, w, b):
            return lax.dot_general(v, w, dn, preferred_element_type=_F32,
                                   precision=lax.Precision.HIGHEST) + b

        h1 = jnp.maximum(dense(pooled_ref[...], f1w_ref[...], f1b_ref[...]), 0.0)
        h2 = jnp.maximum(dense(h1, f2w_ref[...], f2b_ref[...]), 0.0)
        h3 = jnp.maximum(dense(h2, smw_ref[...], smb_ref[...]), 0.0)
        nl = jnp.maximum(dense(news_ref[...], nw_ref[...], nb_ref[...]), 0.0)
        cw = cw_ref[...]
        logit = (jnp.sum(h3 * cw[:, :64], axis=1, keepdims=True)
                 + jnp.sum(nl * cw[:, 64:], axis=1, keepdims=True)
                 + cb_ref[...])
        o_ref[...] = 1.0 / (1.0 + jnp.exp(-logit))


def _fused3(p, cnt2, h, Wl, Wr, b, x, bcol, brow, srow,
            f1w, f1b, f2w, f2b, smw, smb, nw, nb, cw, cb):
    grid = N // R_UPD
    full = lambda shape: pl.BlockSpec(shape, lambda i: tuple(0 for _ in shape))
    return pl.pallas_call(
        _fused3_body,
        grid=(grid,),
        in_specs=[
            pl.BlockSpec((NC, R_UPD, D), lambda i: (0, i, 0)),
            pl.BlockSpec((NC, R_UPD, 1), lambda i: (0, i, 0)),
            pl.BlockSpec((R_UPD, D), lambda i: (i, 0)),
            full((D, D)), full((D, D)), full((1, D)),
            pl.BlockSpec((R_UPD, D), lambda i: (i, 0)),
            pl.BlockSpec((R_UPD, 1), lambda i: (i, 0)),
            pl.BlockSpec((1, 1, R_UPD), lambda i: (i, 0, 0)),
            pl.BlockSpec((1, 1, R_UPD), lambda i: (i, 0, 0)),
            full((D, D)), full((1, D)),
            full((64, D)), full((1, 64)),
            full((64, 64)), full((1, 64)),
            full((64, D)), full((1, 64)),
            full((1, D)), full((1, 1)),
        ],
        out_specs=pl.BlockSpec((G, 1), lambda i: (0, 0)),
        out_shape=jax.ShapeDtypeStruct((G, 1), _F32),
        scratch_shapes=[pltpu.VMEM((G, D), _F32), pltpu.VMEM((G, D), _F32)],
    )(p, cnt2, h, Wl, Wr, b, x, bcol, brow, srow,
      f1w, f1b, f2w, f2b, smw, smb, nw, nb, cw, cb)


def kernel(x, edge_index, batch,
           conv1_Wl, conv1_Wr, conv1_b,
           conv2_Wl, conv2_Wr, conv2_b,
           conv3_Wl, conv3_Wr, conv3_b,
           full1_W, full1_b, full2_W, full2_b,
           softmax_W, softmax_b, lin_news_W, lin_news_b,
           lin_cat_W, lin_cat_b):
    pad_iota = jnp.arange(PAD_E, dtype=jnp.int32)
    src = jnp.concatenate([edge_index[0], pad_iota % N])
    src = src.reshape(NC, NS, NCHUNK, CHUNK)
    dst = jnp.concatenate([edge_index[1], N + pad_iota % (N_PAD - N)])
    dst = dst.reshape(NC, NS, NCHUNK, CHUNK)
    zrows = jnp.zeros((ROWS_PER_TILE, D), _F32)
    zcnt = jnp.zeros((N_PAD,), _F32)

    p1, cnt = _agg_with_cnt(x, src, dst, zrows, zcnt)
    cnt2 = cnt.reshape(NC, N_PAD, 1)
    h1 = _update(p1, cnt2, x, conv1_Wl, conv1_Wr, conv1_b.reshape(1, D))
    p2 = _agg(h1, src, dst, zrows, zcnt)
    h2 = _update(p2, cnt2, h1, conv2_Wl, conv2_Wr, conv2_b.reshape(1, D))
    p3 = _agg(h2, src, dst, zrows, zcnt)

    bf = batch.astype(_F32)
    bcol = bf.reshape(N, 1)
    flat = bf.reshape(1, N)
    sflat = jnp.concatenate([jnp.full((1, 1), -1.0, _F32), flat[:, :-1]], axis=1)
    brow = flat.reshape(N // R_UPD, 1, R_UPD)
    srow = sflat.reshape(N // R_UPD, 1, R_UPD)
    return _fused3(p3, cnt2, h2, conv3_Wl, conv3_Wr, conv3_b.reshape(1, D),
                   x, bcol, brow, srow,
                   full1_W, full1_b.reshape(1, D),
                   full2_W, full2_b.reshape(1, 64),
                   softmax_W, softmax_b.reshape(1, 64),
                   lin_news_W, lin_news_b.reshape(1, 64),
                   lin_cat_W, lin_cat_b.reshape(1, 1))


# speedup vs baseline: 1.0340x; 1.0266x over previous
import functools

import jax
import jax.numpy as jnp
from jax import lax
from jax.experimental import pallas as pl
from jax.experimental.pallas import tpu as pltpu
from jax.experimental.pallas import tpu_sc as plsc

N = 10000
E = 320000
D = 128
G = 32
NC = 2
NS = 16
N_PAD = 10240
ROWS_PER_TILE = N_PAD // NS
CHUNK = 128
NCHUNK = 80
NGRP = 2
GCH = NCHUNK // NGRP
NSUB = 1
E_PAD = NC * NS * NCHUNK * CHUNK
PAD_E = E_PAD - E

_F32 = jnp.float32


def _make_agg(with_cnt: bool):
    mesh = plsc.VectorSubcoreMesh(core_axis_name="c", subcore_axis_name="s")
    if with_cnt:
        out_type = [jax.ShapeDtypeStruct((NC, N_PAD, D), _F32),
                    jax.ShapeDtypeStruct((NC, N_PAD), _F32)]
    else:
        out_type = jax.ShapeDtypeStruct((NC, N_PAD, D), _F32)
    scratch = [
        pltpu.VMEM((GCH, CHUNK), jnp.int32),
        pltpu.VMEM((GCH, CHUNK), jnp.int32),
        pltpu.VMEM((CHUNK, D), _F32),
        pltpu.VMEM((CHUNK, D), _F32),
        pltpu.VMEM((CHUNK,), _F32),
        pltpu.VMEM_SHARED((N_PAD, D), _F32),
        pltpu.VMEM_SHARED((N_PAD,), _F32),
        pltpu.SemaphoreType.DMA,
        pltpu.SemaphoreType.DMA,
    ]

    def body(feats_hbm, edges_hbm, zrows_hbm, zcnt_hbm, *rest):
        if with_cnt:
            out_agg, out_cnt = rest[0], rest[1]
            rest = rest[2:]
        else:
            out_agg = rest[0]
            rest = rest[1:]
        (idx_s, idx_d, rows0, rows1, ones_v, agg_sh, cnt_sh,
         sem0, sem1) = rest
        cid = lax.axis_index("c")
        sid = lax.axis_index("s")
        row0 = sid * ROWS_PER_TILE
        pltpu.sync_copy(zrows_hbm, agg_sh.at[pl.ds(row0, ROWS_PER_TILE)])
        if with_cnt:
            @pl.when(sid == 0)
            def _():
                pltpu.sync_copy(zcnt_hbm, cnt_sh)
            for k in range(CHUNK // 16):
                ones_v[pl.ds(k * 16, 16)] = jnp.ones((16,), _F32)
        plsc.subcore_barrier()

        SUB = CHUNK // NSUB

        def gather_start(i, buf, sem):
            for s in range(NSUB):
                pltpu.async_copy(
                    feats_hbm.at[idx_s.at[i, pl.ds(s * SUB, SUB)]],
                    buf.at[pl.ds(s * SUB, SUB)], sem)

        def gather_wait(i, buf, sem):
            for s in range(NSUB):
                pltpu.make_async_copy(
                    feats_hbm.at[idx_s.at[i, pl.ds(s * SUB, SUB)]],
                    buf.at[pl.ds(s * SUB, SUB)], sem).wait()

        def scatter(i, buf):
            pltpu.sync_copy(buf, agg_sh.at[idx_d.at[i]], add=True)
            if with_cnt:
                pltpu.sync_copy(ones_v, cnt_sh.at[idx_d.at[i]], add=True)

        for g in range(NGRP):
            pltpu.sync_copy(edges_hbm.at[0, cid, sid, pl.ds(g * GCH, GCH)],
                            idx_s)
            pltpu.sync_copy(edges_hbm.at[1, cid, sid, pl.ds(g * GCH, GCH)],
                            idx_d)
            gather_start(0, rows0, sem0)

            def step(j, carry):
                i0 = j * 2
                gather_wait(i0, rows0, sem0)
                gather_start(i0 + 1, rows1, sem1)
                scatter(i0, rows0)
                gather_wait(i0 + 1, rows1, sem1)
                gather_start(i0 + 2, rows0, sem0)
                scatter(i0 + 1, rows1)
                return carry

            lax.fori_loop(0, GCH // 2 - 1, step, 0)
            i0 = GCH - 2
            gather_wait(i0, rows0, sem0)
            gather_start(i0 + 1, rows1, sem1)
            scatter(i0, rows0)
            gather_wait(i0 + 1, rows1, sem1)
            scatter(i0 + 1, rows1)
        plsc.subcore_barrier()
        pltpu.sync_copy(agg_sh.at[pl.ds(row0, ROWS_PER_TILE)],
                        out_agg.at[cid, pl.ds(row0, ROWS_PER_TILE)])
        if with_cnt:
            @pl.when(sid == 0)
            def _():
                pltpu.sync_copy(cnt_sh, out_cnt.at[cid])

    return pl.kernel(body, mesh=mesh, out_type=out_type, scratch_types=scratch)


_agg_with_cnt = _make_agg(True)
_agg = _make_agg(False)


R_UPD = 2000


def _update_body(p_ref, c_ref, h_ref, wl_ref, wr_ref, b_ref, o_ref):
    cnt = c_ref[0] + c_ref[1]
    inv = 1.0 / jnp.maximum(cnt, 1.0)
    mean = (p_ref[0] + p_ref[1]) * inv
    dn = (((1,), (1,)), ((), ()))
    acc = lax.dot_general(mean, wl_ref[...], dn,
                          preferred_element_type=_F32,
                          precision=lax.Precision.HIGHEST)
    acc = acc + lax.dot_general(h_ref[...], wr_ref[...], dn,
                                preferred_element_type=_F32,
                                precision=lax.Precision.HIGHEST)
    o_ref[...] = jnp.maximum(acc + b_ref[...], 0.0)


def _update(p, cnt2, h, Wl, Wr, b):
    grid = N // R_UPD
    return pl.pallas_call(
        _update_body,
        grid=(grid,),
        in_specs=[
            pl.BlockSpec((NC, R_UPD, D), lambda i: (0, i, 0)),
            pl.BlockSpec((NC, R_UPD, 1), lambda i: (0, i, 0)),
            pl.BlockSpec((R_UPD, D), lambda i: (i, 0)),
            pl.BlockSpec((D, D), lambda i: (0, 0)),
            pl.BlockSpec((D, D), lambda i: (0, 0)),
            pl.BlockSpec((1, D), lambda i: (0, 0)),
        ],
        out_specs=pl.BlockSpec((R_UPD, D), lambda i: (i, 0)),
        out_shape=jax.ShapeDtypeStruct((N, D), _F32),
    )(p, cnt2, h, Wl, Wr, b)


NEG = -3.0e38


def _fused3_body(p_ref, c_ref, h_ref, wl_ref, wr_ref, b_ref,
                 x_ref, bcol_ref, brow_ref, srow_ref,
                 f1w_ref, f1b_ref, f2w_ref, f2b_ref, smw_ref, smb_ref,
                 nw_ref, nb_ref, cw_ref, cb_ref, o_ref,
                 pooled_ref, news_ref):
    i = pl.program_id(0)

    @pl.when(i == 0)
    def _():
        pooled_ref[...] = jnp.full((G, D), NEG, _F32)
        news_ref[...] = jnp.zeros((G, D), _F32)

    cnt = c_ref[0] + c_ref[1]
    inv = 1.0 / jnp.maximum(cnt, 1.0)
    mean = (p_ref[0] + p_ref[1]) * inv
    dn = (((1,), (1,)), ((), ()))
    acc = lax.dot_general(mean, wl_ref[...], dn,
                          preferred_element_type=_F32,
                          precision=lax.Precision.HIGHEST)
    acc = acc + lax.dot_general(h_ref[...], wr_ref[...], dn,
                                preferred_element_type=_F32,
                                precision=lax.Precision.HIGHEST)
    hb = jnp.maximum(acc + b_ref[...], 0.0)

    bb = bcol_ref[...]
    blockmax = jnp.concatenate(
        [jnp.max(jnp.where(bb == float(g), hb, NEG), axis=0, keepdims=True)
         for g in range(G)], axis=0)
    pooled_ref[...] = jnp.maximum(pooled_ref[...], blockmax)

    br = brow_ref[0]
    sr = srow_ref[0]
    isroot = jnp.where(br != sr, 1.0, 0.0)
    gid = lax.broadcasted_iota(jnp.int32, (G, 1), 0).astype(_F32)
    onehot = jnp.where(br == gid, 1.0, 0.0) * isroot
    news_ref[...] += lax.dot_general(
        onehot, x_ref[...], (((1,), (0,)), ((), ())),
        preferred_element_type=_F32, precision=lax.Precision.HIGHEST)

    @pl.when(i == (N // R_UPD) - 1)
    def _():
        dn = (((1,), (1,)), ((), ()))

        def dense(v, w, b):
            return lax.dot_general(v, w, dn, preferred_element_type=_F32,
                                   precision=lax.Precision.HIGHEST) + b

        h1 = jnp.maximum(dense(pooled_ref[...], f1w_ref[...], f1b_ref[...]), 0.0)
        h2 = jnp.maximum(dense(h1, f2w_ref[...], f2b_ref[...]), 0.0)
        h3 = jnp.maximum(dense(h2, smw_ref[...], smb_ref[...]), 0.0)
        nl = jnp.maximum(dense(news_ref[...], nw_ref[...], nb_ref[...]), 0.0)
        cw = cw_ref[...]
        logit = (jnp.sum(h3 * cw[:, :64], axis=1, keepdims=True)
                 + jnp.sum(nl * cw[:, 64:], axis=1, keepdims=True)
                 + cb_ref[...])
        o_ref[...] = 1.0 / (1.0 + jnp.exp(-logit))


def _fused3(p, cnt2, h, Wl, Wr, b, x, bcol, brow, srow,
            f1w, f1b, f2w, f2b, smw, smb, nw, nb, cw, cb):
    grid = N // R_UPD
    full = lambda shape: pl.BlockSpec(shape, lambda i: tuple(0 for _ in shape))
    return pl.pallas_call(
        _fused3_body,
        grid=(grid,),
        in_specs=[
            pl.BlockSpec((NC, R_UPD, D), lambda i: (0, i, 0)),
            pl.BlockSpec((NC, R_UPD, 1), lambda i: (0, i, 0)),
            pl.BlockSpec((R_UPD, D), lambda i: (i, 0)),
            full((D, D)), full((D, D)), full((1, D)),
            pl.BlockSpec((R_UPD, D), lambda i: (i, 0)),
            pl.BlockSpec((R_UPD, 1), lambda i: (i, 0)),
            pl.BlockSpec((1, 1, R_UPD), lambda i: (i, 0, 0)),
            pl.BlockSpec((1, 1, R_UPD), lambda i: (i, 0, 0)),
            full((D, D)), full((1, D)),
            full((64, D)), full((1, 64)),
            full((64, 64)), full((1, 64)),
            full((64, D)), full((1, 64)),
            full((1, D)), full((1, 1)),
        ],
        out_specs=pl.BlockSpec((G, 1), lambda i: (0, 0)),
        out_shape=jax.ShapeDtypeStruct((G, 1), _F32),
        scratch_shapes=[pltpu.VMEM((G, D), _F32), pltpu.VMEM((G, D), _F32)],
    )(p, cnt2, h, Wl, Wr, b, x, bcol, brow, srow,
      f1w, f1b, f2w, f2b, smw, smb, nw, nb, cw, cb)


def kernel(x, edge_index, batch,
           conv1_Wl, conv1_Wr, conv1_b,
           conv2_Wl, conv2_Wr, conv2_b,
           conv3_Wl, conv3_Wr, conv3_b,
           full1_W, full1_b, full2_W, full2_b,
           softmax_W, softmax_b, lin_news_W, lin_news_b,
           lin_cat_W, lin_cat_b):
    pad_iota = jnp.arange(PAD_E, dtype=jnp.int32)
    pads = jnp.stack([pad_iota % N, N + pad_iota % (N_PAD - N)])
    edges = jnp.concatenate([edge_index, pads], axis=1)
    edges = edges.reshape(2, NC, NS, NCHUNK, CHUNK)
    zrows = jnp.zeros((ROWS_PER_TILE, D), _F32)
    zcnt = jnp.zeros((N_PAD,), _F32)

    p1, cnt = _agg_with_cnt(x, edges, zrows, zcnt)
    cnt2 = cnt.reshape(NC, N_PAD, 1)
    h1 = _update(p1, cnt2, x, conv1_Wl, conv1_Wr, conv1_b.reshape(1, D))
    p2 = _agg(h1, edges, zrows, zcnt)
    h2 = _update(p2, cnt2, h1, conv2_Wl, conv2_Wr, conv2_b.reshape(1, D))
    p3 = _agg(h2, edges, zrows, zcnt)

    bf = batch.astype(_F32)
    bcol = bf.reshape(N, 1)
    flat = bf.reshape(1, N)
    sflat = jnp.concatenate([jnp.full((1, 1), -1.0, _F32), flat[:, :-1]], axis=1)
    brow = flat.reshape(N // R_UPD, 1, R_UPD)
    srow = sflat.reshape(N // R_UPD, 1, R_UPD)
    return _fused3(p3, cnt2, h2, conv3_Wl, conv3_Wr, conv3_b.reshape(1, D),
                   x, bcol, brow, srow,
                   full1_W, full1_b.reshape(1, D),
                   full2_W, full2_b.reshape(1, 64),
                   softmax_W, softmax_b.reshape(1, 64),
                   lin_news_W, lin_news_b.reshape(1, 64),
                   lin_cat_W, lin_cat_b.reshape(1, 1))


# speedup vs baseline: 1.0353x; 1.0013x over previous
import functools

import jax
import jax.numpy as jnp
import numpy as np
from jax import lax
from jax.experimental import pallas as pl
from jax.experimental.pallas import tpu as pltpu
from jax.experimental.pallas import tpu_sc as plsc

N = 10000
E = 320000
D = 128
G = 32
NC = 2
NS = 16
N_PAD = 10240
ROWS_PER_TILE = N_PAD // NS
CHUNK = 128
NCHUNK = 80
NGRP = 2
GCH = NCHUNK // NGRP
NSUB = 1
E_PAD = NC * NS * NCHUNK * CHUNK
PAD_E = E_PAD - E

_F32 = jnp.float32


def _make_agg(with_cnt: bool):
    mesh = plsc.VectorSubcoreMesh(core_axis_name="c", subcore_axis_name="s")
    if with_cnt:
        out_type = [jax.ShapeDtypeStruct((NC, N_PAD, D), _F32),
                    jax.ShapeDtypeStruct((NC, N_PAD), _F32)]
    else:
        out_type = jax.ShapeDtypeStruct((NC, N_PAD, D), _F32)
    scratch = [
        pltpu.VMEM((GCH, CHUNK), jnp.int32),
        pltpu.VMEM((GCH, CHUNK), jnp.int32),
        pltpu.VMEM((CHUNK, D), _F32),
        pltpu.VMEM((CHUNK, D), _F32),
        pltpu.VMEM((CHUNK,), _F32),
        pltpu.VMEM_SHARED((N_PAD, D), _F32),
        pltpu.VMEM_SHARED((N_PAD,), _F32),
        pltpu.SemaphoreType.DMA,
        pltpu.SemaphoreType.DMA,
    ]

    def body(feats_hbm, edges_hbm, zrows_hbm, zcnt_hbm, *rest):
        if with_cnt:
            out_agg, out_cnt = rest[0], rest[1]
            rest = rest[2:]
        else:
            out_agg = rest[0]
            rest = rest[1:]
        (idx_s, idx_d, rows0, rows1, ones_v, agg_sh, cnt_sh,
         sem0, sem1) = rest
        cid = lax.axis_index("c")
        sid = lax.axis_index("s")
        row0 = sid * ROWS_PER_TILE
        pltpu.sync_copy(zrows_hbm, agg_sh.at[pl.ds(row0, ROWS_PER_TILE)])
        if with_cnt:
            @pl.when(sid == 0)
            def _():
                pltpu.sync_copy(zcnt_hbm, cnt_sh)
            for k in range(CHUNK // 16):
                ones_v[pl.ds(k * 16, 16)] = jnp.ones((16,), _F32)
        plsc.subcore_barrier()

        SUB = CHUNK // NSUB

        def gather_start(i, buf, sem):
            for s in range(NSUB):
                pltpu.async_copy(
                    feats_hbm.at[idx_s.at[i, pl.ds(s * SUB, SUB)]],
                    buf.at[pl.ds(s * SUB, SUB)], sem)

        def gather_wait(i, buf, sem):
            for s in range(NSUB):
                pltpu.make_async_copy(
                    feats_hbm.at[idx_s.at[i, pl.ds(s * SUB, SUB)]],
                    buf.at[pl.ds(s * SUB, SUB)], sem).wait()

        def scatter(i, buf):
            pltpu.sync_copy(buf, agg_sh.at[idx_d.at[i]], add=True)
            if with_cnt:
                pltpu.sync_copy(ones_v, cnt_sh.at[idx_d.at[i]], add=True)

        for g in range(NGRP):
            pltpu.sync_copy(edges_hbm.at[0, cid, sid, pl.ds(g * GCH, GCH)],
                            idx_s)
            pltpu.sync_copy(edges_hbm.at[1, cid, sid, pl.ds(g * GCH, GCH)],
                            idx_d)
            gather_start(0, rows0, sem0)

            def step(j, carry):
                i0 = j * 2
                gather_wait(i0, rows0, sem0)
                gather_start(i0 + 1, rows1, sem1)
                scatter(i0, rows0)
                gather_wait(i0 + 1, rows1, sem1)
                gather_start(i0 + 2, rows0, sem0)
                scatter(i0 + 1, rows1)
                return carry

            lax.fori_loop(0, GCH // 2 - 1, step, 0)
            i0 = GCH - 2
            gather_wait(i0, rows0, sem0)
            gather_start(i0 + 1, rows1, sem1)
            scatter(i0, rows0)
            gather_wait(i0 + 1, rows1, sem1)
            scatter(i0 + 1, rows1)
        plsc.subcore_barrier()
        pltpu.sync_copy(agg_sh.at[pl.ds(row0, ROWS_PER_TILE)],
                        out_agg.at[cid, pl.ds(row0, ROWS_PER_TILE)])
        if with_cnt:
            @pl.when(sid == 0)
            def _():
                pltpu.sync_copy(cnt_sh, out_cnt.at[cid])

    return pl.kernel(body, mesh=mesh, out_type=out_type, scratch_types=scratch)


_agg_with_cnt = _make_agg(True)
_agg = _make_agg(False)


R_UPD = 2000


def _update_body(p_ref, c_ref, h_ref, wl_ref, wr_ref, b_ref, o_ref):
    cnt = c_ref[0] + c_ref[1]
    inv = 1.0 / jnp.maximum(cnt, 1.0)
    mean = (p_ref[0] + p_ref[1]) * inv
    dn = (((1,), (1,)), ((), ()))
    acc = lax.dot_general(mean, wl_ref[...], dn,
                          preferred_element_type=_F32,
                          precision=lax.Precision.HIGHEST)
    acc = acc + lax.dot_general(h_ref[...], wr_ref[...], dn,
                                preferred_element_type=_F32,
                                precision=lax.Precision.HIGHEST)
    o_ref[...] = jnp.maximum(acc + b_ref[...], 0.0)


def _update(p, cnt2, h, Wl, Wr, b):
    grid = N // R_UPD
    return pl.pallas_call(
        _update_body,
        grid=(grid,),
        in_specs=[
            pl.BlockSpec((NC, R_UPD, D), lambda i: (0, i, 0)),
            pl.BlockSpec((NC, R_UPD, 1), lambda i: (0, i, 0)),
            pl.BlockSpec((R_UPD, D), lambda i: (i, 0)),
            pl.BlockSpec((D, D), lambda i: (0, 0)),
            pl.BlockSpec((D, D), lambda i: (0, 0)),
            pl.BlockSpec((1, D), lambda i: (0, 0)),
        ],
        out_specs=pl.BlockSpec((R_UPD, D), lambda i: (i, 0)),
        out_shape=jax.ShapeDtypeStruct((N, D), _F32),
    )(p, cnt2, h, Wl, Wr, b)


NEG = -3.0e38


def _fused3_body(p_ref, c_ref, h_ref, wl_ref, wr_ref, b_ref,
                 x_ref, bcol_ref, brow_ref, srow_ref,
                 f1w_ref, f1b_ref, f2w_ref, f2b_ref, smw_ref, smb_ref,
                 nw_ref, nb_ref, cw_ref, cb_ref, o_ref,
                 pooled_ref, news_ref):
    i = pl.program_id(0)

    @pl.when(i == 0)
    def _():
        pooled_ref[...] = jnp.full((G, D), NEG, _F32)
        news_ref[...] = jnp.zeros((G, D), _F32)

    cnt = c_ref[0] + c_ref[1]
    inv = 1.0 / jnp.maximum(cnt, 1.0)
    mean = (p_ref[0] + p_ref[1]) * inv
    dn = (((1,), (1,)), ((), ()))
    acc = lax.dot_general(mean, wl_ref[...], dn,
                          preferred_element_type=_F32,
                          precision=lax.Precision.HIGHEST)
    acc = acc + lax.dot_general(h_ref[...], wr_ref[...], dn,
                                preferred_element_type=_F32,
                                precision=lax.Precision.HIGHEST)
    hb = jnp.maximum(acc + b_ref[...], 0.0)

    bb = bcol_ref[...]
    blockmax = jnp.concatenate(
        [jnp.max(jnp.where(bb == float(g), hb, NEG), axis=0, keepdims=True)
         for g in range(G)], axis=0)
    pooled_ref[...] = jnp.maximum(pooled_ref[...], blockmax)

    br = brow_ref[0]
    sr = srow_ref[0]
    isroot = jnp.where(br != sr, 1.0, 0.0)
    gid = lax.broadcasted_iota(jnp.int32, (G, 1), 0).astype(_F32)
    onehot = jnp.where(br == gid, 1.0, 0.0) * isroot
    news_ref[...] += lax.dot_general(
        onehot, x_ref[...], (((1,), (0,)), ((), ())),
        preferred_element_type=_F32, precision=lax.Precision.HIGHEST)

    @pl.when(i == (N // R_UPD) - 1)
    def _():
        dn = (((1,), (1,)), ((), ()))

        def dense(v, w, b):
            return lax.dot_general(v, w, dn, preferred_element_type=_F32,
                                   precision=lax.Precision.HIGHEST) + b

        h1 = jnp.maximum(dense(pooled_ref[...], f1w_ref[...], f1b_ref[...]), 0.0)
        h2 = jnp.maximum(dense(h1, f2w_ref[...], f2b_ref[...]), 0.0)
        h3 = jnp.maximum(dense(h2, smw_ref[...], smb_ref[...]), 0.0)
        nl = jnp.maximum(dense(news_ref[...], nw_ref[...], nb_ref[...]), 0.0)
        cw = cw_ref[...]
        logit = (jnp.sum(h3 * cw[:, :64], axis=1, keepdims=True)
                 + jnp.sum(nl * cw[:, 64:], axis=1, keepdims=True)
                 + cb_ref[...])
        o_ref[...] = 1.0 / (1.0 + jnp.exp(-logit))


def _fused3(p, cnt2, h, Wl, Wr, b, x, bcol, brow, srow,
            f1w, f1b, f2w, f2b, smw, smb, nw, nb, cw, cb):
    grid = N // R_UPD
    full = lambda shape: pl.BlockSpec(shape, lambda i: tuple(0 for _ in shape))
    return pl.pallas_call(
        _fused3_body,
        grid=(grid,),
        in_specs=[
            pl.BlockSpec((NC, R_UPD, D), lambda i: (0, i, 0)),
            pl.BlockSpec((NC, R_UPD, 1), lambda i: (0, i, 0)),
            pl.BlockSpec((R_UPD, D), lambda i: (i, 0)),
            full((D, D)), full((D, D)), full((1, D)),
            pl.BlockSpec((R_UPD, D), lambda i: (i, 0)),
            pl.BlockSpec((R_UPD, 1), lambda i: (i, 0)),
            pl.BlockSpec((1, 1, R_UPD), lambda i: (i, 0, 0)),
            pl.BlockSpec((1, 1, R_UPD), lambda i: (i, 0, 0)),
            full((D, D)), full((1, D)),
            full((64, D)), full((1, 64)),
            full((64, 64)), full((1, 64)),
            full((64, D)), full((1, 64)),
            full((1, D)), full((1, 1)),
        ],
        out_specs=pl.BlockSpec((G, 1), lambda i: (0, 0)),
        out_shape=jax.ShapeDtypeStruct((G, 1), _F32),
        scratch_shapes=[pltpu.VMEM((G, D), _F32), pltpu.VMEM((G, D), _F32)],
    )(p, cnt2, h, Wl, Wr, b, x, bcol, brow, srow,
      f1w, f1b, f2w, f2b, smw, smb, nw, nb, cw, cb)


def kernel(x, edge_index, batch,
           conv1_Wl, conv1_Wr, conv1_b,
           conv2_Wl, conv2_Wr, conv2_b,
           conv3_Wl, conv3_Wr, conv3_b,
           full1_W, full1_b, full2_W, full2_b,
           softmax_W, softmax_b, lin_news_W, lin_news_b,
           lin_cat_W, lin_cat_b):
    pad_iota = np.arange(PAD_E, dtype=np.int32)
    pads = jnp.asarray(
        np.stack([pad_iota % N, N + pad_iota % (N_PAD - N)]))
    edges = jnp.concatenate([edge_index, pads], axis=1)
    edges = edges.reshape(2, NC, NS, NCHUNK, CHUNK)
    zrows = jnp.asarray(np.zeros((ROWS_PER_TILE, D), np.float32))
    zcnt = jnp.asarray(np.zeros((N_PAD,), np.float32))

    p1, cnt = _agg_with_cnt(x, edges, zrows, zcnt)
    cnt2 = cnt.reshape(NC, N_PAD, 1)
    h1 = _update(p1, cnt2, x, conv1_Wl, conv1_Wr, conv1_b.reshape(1, D))
    p2 = _agg(h1, edges, zrows, zcnt)
    h2 = _update(p2, cnt2, h1, conv2_Wl, conv2_Wr, conv2_b.reshape(1, D))
    p3 = _agg(h2, edges, zrows, zcnt)

    bf = batch.astype(_F32)
    bcol = batch.astype(jnp.bfloat16).reshape(N, 1)
    flat = bf.reshape(1, N)
    sflat = jnp.concatenate([jnp.full((1, 1), -1.0, _F32), flat[:, :-1]], axis=1)
    brow = flat.reshape(N // R_UPD, 1, R_UPD)
    srow = sflat.reshape(N // R_UPD, 1, R_UPD)
    return _fused3(p3, cnt2, h2, conv3_Wl, conv3_Wr, conv3_b.reshape(1, D),
                   x, bcol, brow, srow,
                   full1_W, full1_b.reshape(1, D),
                   full2_W, full2_b.reshape(1, 64),
                   softmax_W, softmax_b.reshape(1, 64),
                   lin_news_W, lin_news_b.reshape(1, 64),
                   lin_cat_W, lin_cat_b.reshape(1, 1))
